# baseline (speedup 1.0000x reference)
import functools

import jax
import jax.numpy as jnp
from jax import lax
from jax.experimental import pallas as pl
from jax.experimental.pallas import tpu as pltpu
from jax.experimental.pallas import tpu_sc as plsc

BATCH = 16384
N_VOCAB = 1000000
OP_D = 24
SHAPE_D = 8
OUT_D = 32
N_SHAPE_VALS = 2
L = 16
ROW = 128
NCOLS = N_VOCAB // ROW
VTAIL = NCOLS * ROW
PACK_ROWS = NCOLS * OP_D + 8
CP = pltpu.CompilerParams(use_tc_tiling_on_sc=True, needs_layout_passes=False)


DEPTH_IN = 8
DEPTH_PK = 4


def _reformat_body(tab_hbm, out_hbm, in_v, pack_v, sem, osem):
    info = plsc.get_sparse_core_info()
    nc, ns = info.num_cores, info.num_subcores
    nw = nc * ns
    wid = lax.axis_index("s") * nc + lax.axis_index("c")
    cpw = (NCOLS // nw + 1 + DEPTH_IN - 1) // DEPTH_IN * DEPTH_IN
    c0 = wid * (NCOLS // nw + 1)
    lane = lax.iota(jnp.int32, L)

    def clamp(c):
        return jnp.minimum(c, NCOLS - 1)

    def fire_in(c, s):
        cc = pl.multiple_of(clamp(c) * ROW, ROW)
        for r in range(3):
            pltpu.async_copy(tab_hbm.at[pl.ds(r * 8, 8), pl.ds(cc, ROW)],
                             in_v.at[s, r], sem)

    def wait_in(c, s):
        cc = pl.multiple_of(clamp(c) * ROW, ROW)
        for r in range(3):
            pltpu.make_async_copy(
                tab_hbm.at[pl.ds(r * 8, 8), pl.ds(cc, ROW)],
                in_v.at[s, r], sem).wait()

    def shuffle(s, sp):
        for k in range(ROW // L):
            v16 = k * L + lane
            t0 = v16 * OP_D
            for d in range(OP_D):
                t = t0 + d
                x = in_v[s, d // 8, d % 8, pl.ds(k * L, L)]
                plsc.store_scatter(pack_v.at[sp], [t >> 7, t & 127], x)

    def fire_out(c, sp):
        ro = pl.multiple_of(clamp(c) * OP_D, 8)
        pltpu.async_copy(pack_v.at[sp], out_hbm.at[pl.ds(ro, OP_D), :], osem)

    def wait_out(c, sp):
        ro = pl.multiple_of(clamp(c) * OP_D, 8)
        pltpu.make_async_copy(pack_v.at[sp],
                              out_hbm.at[pl.ds(ro, OP_D), :], osem).wait()

    for u in range(DEPTH_IN):
        fire_in(c0 + u, u)

    def step(g, carry):
        cg = c0 + DEPTH_IN * g
        for u in range(DEPTH_IN):
            c = cg + u
            sp = u % DEPTH_PK
            wait_in(c, u)

            @pl.when(DEPTH_IN * g + u >= DEPTH_PK)
            def _():
                wait_out(c - DEPTH_PK, sp)

            shuffle(u, sp)
            fire_out(c, sp)
            fire_in(c + DEPTH_IN, u)
        return carry

    lax.fori_loop(0, cpw // DEPTH_IN, step, 0)
    for u in range(DEPTH_IN):
        wait_in(c0 + cpw + u, u)
    for u in range(DEPTH_PK):
        wait_out(c0 + cpw - DEPTH_PK + u, u % DEPTH_PK)


def _lookup_body(idx_hbm, sv_hbm, pk_hbm, tail_hbm, wb_hbm, out_hbm,
                 idx_v, r0_v, r1_v, sv8_v, rows0_v, rows1_v, rowsT_v, s8_v,
                 tail_v, wb_v, gsem):
    info = plsc.get_sparse_core_info()
    nc, ns = info.num_cores, info.num_subcores
    nw = nc * ns
    bpw = BATCH // nw
    nchunk = bpw // 128
    wid = lax.axis_index("s") * nc + lax.axis_index("c")
    base = wid * bpw
    lane = lax.iota(jnp.int32, L)

    pltpu.sync_copy(idx_hbm.at[wid], idx_v)
    pltpu.sync_copy(tail_hbm, tail_v)

    for j in range(nchunk):
        for k in range(128 // L):
            sl = pl.ds(k * L, L)
            vc = jnp.minimum(idx_v[j, sl], VTAIL - 1)
            r0 = (vc * OP_D) >> 7
            r0_v[j, sl] = r0
            r1_v[j, sl] = r0 + 1

    half = nchunk // 2

    def fire_half(bh):
        gs = []
        for jj in range(half):
            j = bh * half + jj
            gs.append(pltpu.async_copy(
                pk_hbm.at[r0_v.at[j]],
                rows0_v.at[pl.ds(jj * 128, 128), :], gsem))
            gs.append(pltpu.async_copy(
                pk_hbm.at[r1_v.at[j]],
                rows1_v.at[pl.ds(jj * 128, 128), :], gsem))
        return gs

    g0 = fire_half(0)

    pltpu.sync_copy(sv_hbm.at[:, pl.ds(base, bpw)], sv8_v)
    pltpu.sync_copy(wb_hbm, wb_v)
    w_bc = [plsc.load_gather(wb_v, [jnp.full((L,), 1 + k, jnp.int32)])
            for k in range(N_SHAPE_VALS * SHAPE_D)]
    b_bc = [plsc.load_gather(wb_v, [jnp.full((L,), 17 + j, jnp.int32)])
            for j in range(SHAPE_D)]

    def chunk(i, carry):
        sl = pl.ds(i * L, L)
        sv0 = sv8_v[0, sl]
        sv1 = sv8_v[1, sl]
        for j in range(SHAPE_D):
            s8_v[j, sl] = sv0 * w_bc[j] + sv1 * w_bc[SHAPE_D + j] + b_bc[j]
        return carry

    lax.fori_loop(0, bpw // L, chunk, 0)
    pltpu.sync_copy(s8_v, out_hbm.at[pl.ds(OP_D, SHAPE_D), pl.ds(base, bpw)])

    def extract_half(bh):
        for jj in range(half):
            j = bh * half + jj

            def extract(k, carry, j=j, jj=jj):
                n = jj * 128 + k * L + lane
                v = idx_v[j, pl.ds(k * L, L)]
                vc = jnp.minimum(v, VTAIL - 1)
                off = (vc * OP_D) & 127
                istail = v >= VTAIL
                tloc = jnp.minimum(jnp.maximum(v - VTAIL, 0), 63)
                for d in range(OP_D):
                    pos = off + d
                    x0 = plsc.load_gather(rows0_v,
                                          [n, jnp.minimum(pos, 127)])
                    x1 = plsc.load_gather(rows1_v,
                                          [n, jnp.maximum(pos - 128, 0)])
                    xt = plsc.load_gather(
                        tail_v, [tloc, jnp.full((L,), d, jnp.int32)])
                    x = jnp.where(istail, xt, jnp.where(pos < 128, x0, x1))
                    rowsT_v[d, pl.ds(j * 128 + k * L, L)] = x
                return carry

            lax.fori_loop(0, 128 // L, extract, 0)

    for g in g0:
        g.wait()
    extract_half(0)
    g1 = fire_half(1)
    for g in g1:
        g.wait()
    extract_half(1)
    pltpu.sync_copy(rowsT_v, out_hbm.at[pl.ds(0, OP_D), pl.ds(base, bpw)])


def kernel(node_inds, shape_vals, op_table, lin_W, lin_b):
    info = plsc.get_sparse_core_info()
    nw = info.num_cores * info.num_subcores
    bpw = BATCH // nw
    mesh = plsc.VectorSubcoreMesh(core_axis_name="c", subcore_axis_name="s")

    reformat = functools.partial(
        pl.kernel, mesh=mesh, compiler_params=CP,
        out_type=jax.ShapeDtypeStruct((PACK_ROWS, ROW), jnp.float32),
        scratch_types=[
            pltpu.VMEM((DEPTH_IN, 3, 8, ROW), jnp.float32),
            pltpu.VMEM((DEPTH_PK, OP_D, ROW), jnp.float32),
            pltpu.SemaphoreType.DMA,
            pltpu.SemaphoreType.DMA,
        ],
    )(_reformat_body)

    lookup = functools.partial(
        pl.kernel, mesh=mesh, compiler_params=CP,
        out_type=jax.ShapeDtypeStruct((OUT_D, BATCH), jnp.float32),
        scratch_types=[
            pltpu.VMEM((8, 128), jnp.int32),
            pltpu.VMEM((8, 128), jnp.int32),
            pltpu.VMEM((8, 128), jnp.int32),
            pltpu.VMEM((8, bpw), jnp.float32),
            pltpu.VMEM((bpw // 2, ROW), jnp.float32),
            pltpu.VMEM((bpw // 2, ROW), jnp.float32),
            pltpu.VMEM((OP_D, bpw), jnp.float32),
            pltpu.VMEM((SHAPE_D, bpw), jnp.float32),
            pltpu.VMEM((64, ROW), jnp.float32),
            pltpu.VMEM((32,), jnp.float32),
            pltpu.SemaphoreType.DMA,
        ],
    )(_lookup_body)

    packed = reformat(op_table.T)
    wb = jnp.concatenate(
        [jnp.zeros((1,), jnp.float32), lin_W.reshape(-1), lin_b,
         jnp.zeros((7,), jnp.float32)])
    idxp = jnp.pad(node_inds.astype(jnp.int32).reshape(nw, bpw // 128, 128),
                   ((0, 0), (0, 8 - bpw // 128), (0, 0)))
    svp = jnp.pad(shape_vals.T, ((0, 8 - N_SHAPE_VALS), (0, 0)))
    tailp = jnp.pad(op_table[VTAIL:], ((0, 0), (0, ROW - OP_D)))
    out_t = lookup(idxp, svp, packed, tailp, wb)
    return out_t.T


# speedup vs baseline: 1.0621x; 1.0621x over previous
import functools

import jax
import jax.numpy as jnp
from jax import lax
from jax.experimental import pallas as pl
from jax.experimental.pallas import tpu as pltpu
from jax.experimental.pallas import tpu_sc as plsc

BATCH = 16384
N_VOCAB = 1000000
OP_D = 24
SHAPE_D = 8
OUT_D = 32
N_SHAPE_VALS = 2
L = 16
ROW = 128
NCOLS = N_VOCAB // ROW
VTAIL = NCOLS * ROW
PACK_ROWS = NCOLS * OP_D + 8
CP = pltpu.CompilerParams(use_tc_tiling_on_sc=True, needs_layout_passes=False)


DEPTH_IN = 4
DEPTH_PK = 4


def _reformat_body(tab_hbm, out_hbm, in_v, pack_v, sem, osem):
    info = plsc.get_sparse_core_info()
    nc, ns = info.num_cores, info.num_subcores
    nw = nc * ns
    wid = lax.axis_index("s") * nc + lax.axis_index("c")
    cpw = (NCOLS // nw + 1 + DEPTH_IN - 1) // DEPTH_IN * DEPTH_IN
    c0 = wid * (NCOLS // nw + 1)
    lane = lax.iota(jnp.int32, L)

    def clamp(c):
        return jnp.minimum(c, NCOLS - 1)

    def fire_in(c, s):
        cc = pl.multiple_of(clamp(c) * ROW, ROW)
        for r in range(3):
            pltpu.async_copy(tab_hbm.at[pl.ds(r * 8, 8), pl.ds(cc, ROW)],
                             in_v.at[s, r], sem)

    def wait_in(c, s):
        cc = pl.multiple_of(clamp(c) * ROW, ROW)
        for r in range(3):
            pltpu.make_async_copy(
                tab_hbm.at[pl.ds(r * 8, 8), pl.ds(cc, ROW)],
                in_v.at[s, r], sem).wait()

    def shuffle(s, sp):
        for k in range(ROW // L):
            v16 = k * L + lane
            t0 = v16 * OP_D
            for d in range(OP_D):
                t = t0 + d
                x = in_v[s, d // 8, d % 8, pl.ds(k * L, L)]
                plsc.store_scatter(pack_v.at[sp], [t >> 7, t & 127], x)

    def fire_out(c, sp):
        ro = pl.multiple_of(clamp(c) * OP_D, 8)
        pltpu.async_copy(pack_v.at[sp], out_hbm.at[pl.ds(ro, OP_D), :], osem)

    def wait_out(c, sp):
        ro = pl.multiple_of(clamp(c) * OP_D, 8)
        pltpu.make_async_copy(pack_v.at[sp],
                              out_hbm.at[pl.ds(ro, OP_D), :], osem).wait()

    for u in range(DEPTH_IN):
        fire_in(c0 + u, u)

    def step(g, carry):
        cg = c0 + DEPTH_IN * g
        for u in range(DEPTH_IN):
            c = cg + u
            sp = u % DEPTH_PK
            wait_in(c, u)

            @pl.when(DEPTH_IN * g + u >= DEPTH_PK)
            def _():
                wait_out(c - DEPTH_PK, sp)

            shuffle(u, sp)
            fire_out(c, sp)
            fire_in(c + DEPTH_IN, u)
        return carry

    lax.fori_loop(0, cpw // DEPTH_IN, step, 0)
    for u in range(DEPTH_IN):
        wait_in(c0 + cpw + u, u)
    for u in range(DEPTH_PK):
        wait_out(c0 + cpw - DEPTH_PK + u, u % DEPTH_PK)


def _lookup_body(idx_hbm, sv_hbm, pk_hbm, tail_hbm, wb_hbm, out_hbm,
                 idx_v, r0_v, r1_v, sv8_v, rows0_v, rows1_v, rowsT_v, s8_v,
                 tail_v, wb_v, gsem):
    info = plsc.get_sparse_core_info()
    nc, ns = info.num_cores, info.num_subcores
    nw = nc * ns
    bpw = BATCH // nw
    nchunk = bpw // 128
    wid = lax.axis_index("s") * nc + lax.axis_index("c")
    base = wid * bpw
    lane = lax.iota(jnp.int32, L)

    pltpu.sync_copy(idx_hbm.at[wid], idx_v)
    pltpu.sync_copy(tail_hbm, tail_v)

    for j in range(nchunk):
        for k in range(128 // L):
            sl = pl.ds(k * L, L)
            vc = jnp.minimum(idx_v[j, sl], VTAIL - 1)
            r0 = (vc * OP_D) >> 7
            r0_v[j, sl] = r0
            r1_v[j, sl] = r0 + 1

    half = nchunk // 2

    def fire_half(bh):
        gs = []
        for jj in range(half):
            j = bh * half + jj
            gs.append(pltpu.async_copy(
                pk_hbm.at[r0_v.at[j]],
                rows0_v.at[pl.ds(jj * 128, 128), :], gsem))
            gs.append(pltpu.async_copy(
                pk_hbm.at[r1_v.at[j]],
                rows1_v.at[pl.ds(jj * 128, 128), :], gsem))
        return gs

    g0 = fire_half(0)

    pltpu.sync_copy(sv_hbm.at[:, pl.ds(base, bpw)], sv8_v)
    pltpu.sync_copy(wb_hbm, wb_v)
    w_bc = [plsc.load_gather(wb_v, [jnp.full((L,), 1 + k, jnp.int32)])
            for k in range(N_SHAPE_VALS * SHAPE_D)]
    b_bc = [plsc.load_gather(wb_v, [jnp.full((L,), 17 + j, jnp.int32)])
            for j in range(SHAPE_D)]

    def chunk(i, carry):
        sl = pl.ds(i * L, L)
        sv0 = sv8_v[0, sl]
        sv1 = sv8_v[1, sl]
        for j in range(SHAPE_D):
            s8_v[j, sl] = sv0 * w_bc[j] + sv1 * w_bc[SHAPE_D + j] + b_bc[j]
        return carry

    lax.fori_loop(0, bpw // L, chunk, 0)
    pltpu.sync_copy(s8_v, out_hbm.at[pl.ds(OP_D, SHAPE_D), pl.ds(base, bpw)])

    def extract_half(bh):
        for jj in range(half):
            j = bh * half + jj

            def extract(k, carry, j=j, jj=jj):
                n = jj * 128 + k * L + lane
                v = idx_v[j, pl.ds(k * L, L)]
                vc = jnp.minimum(v, VTAIL - 1)
                off = (vc * OP_D) & 127
                istail = v >= VTAIL
                tloc = jnp.minimum(jnp.maximum(v - VTAIL, 0), 63)
                for d in range(OP_D):
                    pos = off + d
                    x0 = plsc.load_gather(rows0_v,
                                          [n, jnp.minimum(pos, 127)])
                    x1 = plsc.load_gather(rows1_v,
                                          [n, jnp.maximum(pos - 128, 0)])
                    xt = plsc.load_gather(
                        tail_v, [tloc, jnp.full((L,), d, jnp.int32)])
                    x = jnp.where(istail, xt, jnp.where(pos < 128, x0, x1))
                    rowsT_v[d, pl.ds(j * 128 + k * L, L)] = x
                return carry

            lax.fori_loop(0, 128 // L, extract, 0)

    for g in g0:
        g.wait()
    extract_half(0)
    g1 = fire_half(1)
    for g in g1:
        g.wait()
    extract_half(1)
    pltpu.sync_copy(rowsT_v, out_hbm.at[pl.ds(0, OP_D), pl.ds(base, bpw)])


def kernel(node_inds, shape_vals, op_table, lin_W, lin_b):
    info = plsc.get_sparse_core_info()
    nw = info.num_cores * info.num_subcores
    bpw = BATCH // nw
    mesh = plsc.VectorSubcoreMesh(core_axis_name="c", subcore_axis_name="s")

    reformat = functools.partial(
        pl.kernel, mesh=mesh, compiler_params=CP,
        out_type=jax.ShapeDtypeStruct((PACK_ROWS, ROW), jnp.float32),
        scratch_types=[
            pltpu.VMEM((DEPTH_IN, 3, 8, ROW), jnp.float32),
            pltpu.VMEM((DEPTH_PK, OP_D, ROW), jnp.float32),
            pltpu.SemaphoreType.DMA,
            pltpu.SemaphoreType.DMA,
        ],
    )(_reformat_body)

    lookup = functools.partial(
        pl.kernel, mesh=mesh, compiler_params=CP,
        out_type=jax.ShapeDtypeStruct((OUT_D, BATCH), jnp.float32),
        scratch_types=[
            pltpu.VMEM((8, 128), jnp.int32),
            pltpu.VMEM((8, 128), jnp.int32),
            pltpu.VMEM((8, 128), jnp.int32),
            pltpu.VMEM((8, bpw), jnp.float32),
            pltpu.VMEM((bpw // 2, ROW), jnp.float32),
            pltpu.VMEM((bpw // 2, ROW), jnp.float32),
            pltpu.VMEM((OP_D, bpw), jnp.float32),
            pltpu.VMEM((SHAPE_D, bpw), jnp.float32),
            pltpu.VMEM((64, ROW), jnp.float32),
            pltpu.VMEM((32,), jnp.float32),
            pltpu.SemaphoreType.DMA,
        ],
    )(_lookup_body)

    packed = reformat(op_table.T)
    wb = jnp.concatenate(
        [jnp.zeros((1,), jnp.float32), lin_W.reshape(-1), lin_b,
         jnp.zeros((7,), jnp.float32)])
    idxp = jnp.pad(node_inds.astype(jnp.int32).reshape(nw, bpw // 128, 128),
                   ((0, 0), (0, 8 - bpw // 128), (0, 0)))
    svp = jnp.pad(shape_vals.T, ((0, 8 - N_SHAPE_VALS), (0, 0)))
    tailp = jnp.pad(op_table[VTAIL:], ((0, 0), (0, ROW - OP_D)))
    out_t = lookup(idxp, svp, packed, tailp, wb)
    return out_t.T


# speedup vs baseline: 1.1728x; 1.1042x over previous
import functools

import jax
import jax.numpy as jnp
from jax import lax
from jax.experimental import pallas as pl
from jax.experimental.pallas import tpu as pltpu
from jax.experimental.pallas import tpu_sc as plsc

BATCH = 16384
N_VOCAB = 1000000
OP_D = 24
SHAPE_D = 8
OUT_D = 32
N_SHAPE_VALS = 2
L = 16
ROW = 128
NCOLS = N_VOCAB // ROW
VTAIL = NCOLS * ROW
PACK_ROWS = NCOLS * OP_D + 8
CP = pltpu.CompilerParams(use_tc_tiling_on_sc=True, needs_layout_passes=False)


def _reformat_body(tab_hbm, out_hbm, in_v, pack_v, sem, osem):
    info = plsc.get_sparse_core_info()
    nc, ns = info.num_cores, info.num_subcores
    nw = nc * ns
    wid = lax.axis_index("s") * nc + lax.axis_index("c")
    cpw = (NCOLS + nw - 1) // nw + 1
    cpw = cpw + (cpw % 2)
    c0 = wid * (NCOLS // nw + 1)
    lane = lax.iota(jnp.int32, L)

    def clamp(c):
        return jnp.minimum(c, NCOLS - 1)

    def fire_in(c, s):
        cc = pl.multiple_of(clamp(c) * ROW, ROW)
        for r in range(3):
            pltpu.async_copy(tab_hbm.at[pl.ds(r * 8, 8), pl.ds(cc, ROW)],
                             in_v.at[s, r], sem)

    def wait_in(c, s):
        cc = pl.multiple_of(clamp(c) * ROW, ROW)
        for r in range(3):
            pltpu.make_async_copy(
                tab_hbm.at[pl.ds(r * 8, 8), pl.ds(cc, ROW)],
                in_v.at[s, r], sem).wait()

    def shuffle(s):
        for k in range(ROW // L):
            v16 = k * L + lane
            t0 = v16 * OP_D
            for d in range(OP_D):
                t = t0 + d
                x = in_v[s, d // 8, d % 8, pl.ds(k * L, L)]
                plsc.store_scatter(pack_v.at[s], [t >> 7, t & 127], x)

    def fire_out(c, s):
        ro = pl.multiple_of(clamp(c) * OP_D, 8)
        pltpu.async_copy(pack_v.at[s], out_hbm.at[pl.ds(ro, OP_D), :], osem)

    def wait_out(c, s):
        ro = pl.multiple_of(clamp(c) * OP_D, 8)
        pltpu.make_async_copy(pack_v.at[s],
                              out_hbm.at[pl.ds(ro, OP_D), :], osem).wait()

    fire_in(c0, 0)
    fire_in(c0 + 1, 1)

    def step(g, carry):
        c = c0 + 2 * g
        wait_in(c, 0)
        shuffle(0)
        fire_out(c, 0)
        fire_in(c + 2, 0)
        wait_in(c + 1, 1)
        shuffle(1)
        fire_out(c + 1, 1)
        fire_in(c + 3, 1)
        wait_out(c, 0)
        wait_out(c + 1, 1)
        return carry

    lax.fori_loop(0, cpw // 2, step, 0)
    wait_in(c0 + cpw, 0)
    wait_in(c0 + cpw + 1, 1)


def _lookup_body(idx_hbm, sv_hbm, pk_hbm, tail_hbm, wb_hbm, out_hbm,
                 idx_v, r0_v, r1_v, sv8_v, rows0_v, rows1_v, rowsT_v, s8_v,
                 tail_v, wb_v, gsem):
    info = plsc.get_sparse_core_info()
    nc, ns = info.num_cores, info.num_subcores
    nw = nc * ns
    bpw = BATCH // nw
    nchunk = bpw // 128
    wid = lax.axis_index("s") * nc + lax.axis_index("c")
    base = wid * bpw
    lane = lax.iota(jnp.int32, L)

    pltpu.sync_copy(idx_hbm.at[wid], idx_v)
    pltpu.sync_copy(tail_hbm, tail_v)

    for j in range(nchunk):
        for k in range(128 // L):
            sl = pl.ds(k * L, L)
            vc = jnp.minimum(idx_v[j, sl], VTAIL - 1)
            r0 = (vc * OP_D) >> 7
            r0_v[j, sl] = r0
            r1_v[j, sl] = r0 + 1

    half = nchunk // 2

    def fire_half(bh):
        gs = []
        for jj in range(half):
            j = bh * half + jj
            gs.append(pltpu.async_copy(
                pk_hbm.at[r0_v.at[j]],
                rows0_v.at[pl.ds(jj * 128, 128), :], gsem))
            gs.append(pltpu.async_copy(
                pk_hbm.at[r1_v.at[j]],
                rows1_v.at[pl.ds(jj * 128, 128), :], gsem))
        return gs

    g0 = fire_half(0)

    pltpu.sync_copy(sv_hbm.at[:, pl.ds(base, bpw)], sv8_v)
    pltpu.sync_copy(wb_hbm, wb_v)
    w_bc = [plsc.load_gather(wb_v, [jnp.full((L,), 1 + k, jnp.int32)])
            for k in range(N_SHAPE_VALS * SHAPE_D)]
    b_bc = [plsc.load_gather(wb_v, [jnp.full((L,), 17 + j, jnp.int32)])
            for j in range(SHAPE_D)]

    def chunk(i, carry):
        sl = pl.ds(i * L, L)
        sv0 = sv8_v[0, sl]
        sv1 = sv8_v[1, sl]
        for j in range(SHAPE_D):
            s8_v[j, sl] = sv0 * w_bc[j] + sv1 * w_bc[SHAPE_D + j] + b_bc[j]
        return carry

    lax.fori_loop(0, bpw // L, chunk, 0)
    pltpu.sync_copy(s8_v, out_hbm.at[pl.ds(OP_D, SHAPE_D), pl.ds(base, bpw)])

    def extract_half(bh):
        for jj in range(half):
            j = bh * half + jj

            def extract(k, carry, j=j, jj=jj):
                n = jj * 128 + k * L + lane
                v = idx_v[j, pl.ds(k * L, L)]
                vc = jnp.minimum(v, VTAIL - 1)
                off = (vc * OP_D) & 127
                istail = v >= VTAIL
                tloc = jnp.minimum(jnp.maximum(v - VTAIL, 0), 63)
                for d in range(OP_D):
                    pos = off + d
                    x0 = plsc.load_gather(rows0_v,
                                          [n, jnp.minimum(pos, 127)])
                    x1 = plsc.load_gather(rows1_v,
                                          [n, jnp.maximum(pos - 128, 0)])
                    xt = plsc.load_gather(
                        tail_v, [tloc, jnp.full((L,), d, jnp.int32)])
                    x = jnp.where(istail, xt, jnp.where(pos < 128, x0, x1))
                    rowsT_v[d, pl.ds(j * 128 + k * L, L)] = x
                return carry

            lax.fori_loop(0, 128 // L, extract, 0)

    for g in g0:
        g.wait()
    extract_half(0)
    g1 = fire_half(1)
    for g in g1:
        g.wait()
    extract_half(1)
    pltpu.sync_copy(rowsT_v, out_hbm.at[pl.ds(0, OP_D), pl.ds(base, bpw)])


def kernel(node_inds, shape_vals, op_table, lin_W, lin_b):
    info = plsc.get_sparse_core_info()
    nw = info.num_cores * info.num_subcores
    bpw = BATCH // nw
    mesh = plsc.VectorSubcoreMesh(core_axis_name="c", subcore_axis_name="s")

    reformat = functools.partial(
        pl.kernel, mesh=mesh, compiler_params=CP,
        out_type=jax.ShapeDtypeStruct((PACK_ROWS, ROW), jnp.float32),
        scratch_types=[
            pltpu.VMEM((2, 3, 8, ROW), jnp.float32),
            pltpu.VMEM((2, OP_D, ROW), jnp.float32),
            pltpu.SemaphoreType.DMA,
            pltpu.SemaphoreType.DMA,
        ],
    )(_reformat_body)

    lookup = functools.partial(
        pl.kernel, mesh=mesh, compiler_params=CP,
        out_type=jax.ShapeDtypeStruct((OUT_D, BATCH), jnp.float32),
        scratch_types=[
            pltpu.VMEM((8, 128), jnp.int32),
            pltpu.VMEM((8, 128), jnp.int32),
            pltpu.VMEM((8, 128), jnp.int32),
            pltpu.VMEM((8, bpw), jnp.float32),
            pltpu.VMEM((bpw // 2, ROW), jnp.float32),
            pltpu.VMEM((bpw // 2, ROW), jnp.float32),
            pltpu.VMEM((OP_D, bpw), jnp.float32),
            pltpu.VMEM((SHAPE_D, bpw), jnp.float32),
            pltpu.VMEM((64, ROW), jnp.float32),
            pltpu.VMEM((32,), jnp.float32),
            pltpu.SemaphoreType.DMA,
        ],
    )(_lookup_body)

    packed = reformat(op_table.T)
    wb = jnp.concatenate(
        [jnp.zeros((1,), jnp.float32), lin_W.reshape(-1), lin_b,
         jnp.zeros((7,), jnp.float32)])
    idxp = jnp.pad(node_inds.astype(jnp.int32).reshape(nw, bpw // 128, 128),
                   ((0, 0), (0, 8 - bpw // 128), (0, 0)))
    svp = jnp.pad(shape_vals.T, ((0, 8 - N_SHAPE_VALS), (0, 0)))
    tailp = jnp.pad(op_table[VTAIL:], ((0, 0), (0, ROW - OP_D)))
    out_t = lookup(idxp, svp, packed, tailp, wb)
    return out_t.T


# speedup vs baseline: 1.1871x; 1.0122x over previous
import functools

import jax
import jax.numpy as jnp
from jax import lax
from jax.experimental import pallas as pl
from jax.experimental.pallas import tpu as pltpu
from jax.experimental.pallas import tpu_sc as plsc

BATCH = 16384
N_VOCAB = 1000000
OP_D = 24
SHAPE_D = 8
OUT_D = 32
N_SHAPE_VALS = 2
L = 16
ROW = 128
NCOLS = N_VOCAB // ROW
VTAIL = NCOLS * ROW
PACK_ROWS = NCOLS * OP_D + 8
CP = pltpu.CompilerParams(use_tc_tiling_on_sc=True, needs_layout_passes=False)


def _reformat_body(tab_hbm, out_hbm, in_v, pack_v, sem, osem):
    info = plsc.get_sparse_core_info()
    nc, ns = info.num_cores, info.num_subcores
    nw = nc * ns
    wid = lax.axis_index("s") * nc + lax.axis_index("c")
    cpw = (NCOLS + nw - 1) // nw + 1
    cpw = cpw + (cpw % 2)
    c0 = wid * (NCOLS // nw + 1)
    lane = lax.iota(jnp.int32, L)

    def clamp(c):
        return jnp.minimum(c, NCOLS - 1)

    def fire_in(c, s):
        cc = pl.multiple_of(clamp(c) * ROW, ROW)
        for r in range(3):
            pltpu.async_copy(tab_hbm.at[pl.ds(r * 8, 8), pl.ds(cc, ROW)],
                             in_v.at[s, r], sem)

    def wait_in(c, s):
        cc = pl.multiple_of(clamp(c) * ROW, ROW)
        for r in range(3):
            pltpu.make_async_copy(
                tab_hbm.at[pl.ds(r * 8, 8), pl.ds(cc, ROW)],
                in_v.at[s, r], sem).wait()

    def shuffle(s):
        for k in range(ROW // L):
            v16 = k * L + lane
            t0 = v16 * OP_D
            for d in range(OP_D):
                t = t0 + d
                x = in_v[s, d // 8, d % 8, pl.ds(k * L, L)]
                plsc.store_scatter(pack_v.at[s], [t >> 7, t & 127], x)

    def fire_out(c, s):
        ro = pl.multiple_of(clamp(c) * OP_D, 8)
        pltpu.async_copy(pack_v.at[s], out_hbm.at[pl.ds(ro, OP_D), :], osem)

    def wait_out(c, s):
        ro = pl.multiple_of(clamp(c) * OP_D, 8)
        pltpu.make_async_copy(pack_v.at[s],
                              out_hbm.at[pl.ds(ro, OP_D), :], osem).wait()

    fire_in(c0, 0)
    fire_in(c0 + 1, 1)

    def step(g, carry):
        c = c0 + 2 * g
        wait_in(c, 0)

        @pl.when(g > 0)
        def _():
            wait_out(c - 2, 0)

        shuffle(0)
        fire_out(c, 0)
        fire_in(c + 2, 0)
        wait_in(c + 1, 1)

        @pl.when(g > 0)
        def _():
            wait_out(c - 1, 1)

        shuffle(1)
        fire_out(c + 1, 1)
        fire_in(c + 3, 1)
        return carry

    lax.fori_loop(0, cpw // 2, step, 0)
    wait_in(c0 + cpw, 0)
    wait_in(c0 + cpw + 1, 1)
    wait_out(c0 + cpw - 2, 0)
    wait_out(c0 + cpw - 1, 1)


def _lookup_body(idx_hbm, sv_hbm, pk_hbm, tail_hbm, wb_hbm, out_hbm,
                 idx_v, r0_v, r1_v, sv8_v, rows0_v, rows1_v, rowsT_v, s8_v,
                 tail_v, wb_v, gsem):
    info = plsc.get_sparse_core_info()
    nc, ns = info.num_cores, info.num_subcores
    nw = nc * ns
    bpw = BATCH // nw
    nchunk = bpw // 128
    wid = lax.axis_index("s") * nc + lax.axis_index("c")
    base = wid * bpw
    lane = lax.iota(jnp.int32, L)

    pltpu.sync_copy(idx_hbm.at[wid], idx_v)
    pltpu.sync_copy(tail_hbm, tail_v)

    for j in range(nchunk):
        for k in range(128 // L):
            sl = pl.ds(k * L, L)
            vc = jnp.minimum(idx_v[j, sl], VTAIL - 1)
            r0 = (vc * OP_D) >> 7
            r0_v[j, sl] = r0
            r1_v[j, sl] = r0 + 1

    half = nchunk // 2

    def fire_half(bh):
        gs = []
        for jj in range(half):
            j = bh * half + jj
            gs.append(pltpu.async_copy(
                pk_hbm.at[r0_v.at[j]],
                rows0_v.at[pl.ds(jj * 128, 128), :], gsem))
            gs.append(pltpu.async_copy(
                pk_hbm.at[r1_v.at[j]],
                rows1_v.at[pl.ds(jj * 128, 128), :], gsem))
        return gs

    g0 = fire_half(0)

    pltpu.sync_copy(sv_hbm.at[:, pl.ds(base, bpw)], sv8_v)
    pltpu.sync_copy(wb_hbm, wb_v)
    w_bc = [plsc.load_gather(wb_v, [jnp.full((L,), 1 + k, jnp.int32)])
            for k in range(N_SHAPE_VALS * SHAPE_D)]
    b_bc = [plsc.load_gather(wb_v, [jnp.full((L,), 17 + j, jnp.int32)])
            for j in range(SHAPE_D)]

    def chunk(i, carry):
        sl = pl.ds(i * L, L)
        sv0 = sv8_v[0, sl]
        sv1 = sv8_v[1, sl]
        for j in range(SHAPE_D):
            s8_v[j, sl] = sv0 * w_bc[j] + sv1 * w_bc[SHAPE_D + j] + b_bc[j]
        return carry

    lax.fori_loop(0, bpw // L, chunk, 0)
    pltpu.sync_copy(s8_v, out_hbm.at[pl.ds(OP_D, SHAPE_D), pl.ds(base, bpw)])

    def extract_half(bh):
        for jj in range(half):
            j = bh * half + jj

            def extract(k, carry, j=j, jj=jj):
                n = jj * 128 + k * L + lane
                v = idx_v[j, pl.ds(k * L, L)]
                vc = jnp.minimum(v, VTAIL - 1)
                off = (vc * OP_D) & 127
                istail = v >= VTAIL
                tloc = jnp.minimum(jnp.maximum(v - VTAIL, 0), 63)
                for d in range(OP_D):
                    pos = off + d
                    x0 = plsc.load_gather(rows0_v,
                                          [n, jnp.minimum(pos, 127)])
                    x1 = plsc.load_gather(rows1_v,
                                          [n, jnp.maximum(pos - 128, 0)])
                    xt = plsc.load_gather(
                        tail_v, [tloc, jnp.full((L,), d, jnp.int32)])
                    x = jnp.where(istail, xt, jnp.where(pos < 128, x0, x1))
                    rowsT_v[d, pl.ds(j * 128 + k * L, L)] = x
                return carry

            lax.fori_loop(0, 128 // L, extract, 0)

    for g in g0:
        g.wait()
    extract_half(0)
    g1 = fire_half(1)
    for g in g1:
        g.wait()
    extract_half(1)
    pltpu.sync_copy(rowsT_v, out_hbm.at[pl.ds(0, OP_D), pl.ds(base, bpw)])


def kernel(node_inds, shape_vals, op_table, lin_W, lin_b):
    info = plsc.get_sparse_core_info()
    nw = info.num_cores * info.num_subcores
    bpw = BATCH // nw
    mesh = plsc.VectorSubcoreMesh(core_axis_name="c", subcore_axis_name="s")

    reformat = functools.partial(
        pl.kernel, mesh=mesh, compiler_params=CP,
        out_type=jax.ShapeDtypeStruct((PACK_ROWS, ROW), jnp.float32),
        scratch_types=[
            pltpu.VMEM((2, 3, 8, ROW), jnp.float32),
            pltpu.VMEM((2, OP_D, ROW), jnp.float32),
            pltpu.SemaphoreType.DMA,
            pltpu.SemaphoreType.DMA,
        ],
    )(_reformat_body)

    lookup = functools.partial(
        pl.kernel, mesh=mesh, compiler_params=CP,
        out_type=jax.ShapeDtypeStruct((OUT_D, BATCH), jnp.float32),
        scratch_types=[
            pltpu.VMEM((8, 128), jnp.int32),
            pltpu.VMEM((8, 128), jnp.int32),
            pltpu.VMEM((8, 128), jnp.int32),
            pltpu.VMEM((8, bpw), jnp.float32),
            pltpu.VMEM((bpw // 2, ROW), jnp.float32),
            pltpu.VMEM((bpw // 2, ROW), jnp.float32),
            pltpu.VMEM((OP_D, bpw), jnp.float32),
            pltpu.VMEM((SHAPE_D, bpw), jnp.float32),
            pltpu.VMEM((64, ROW), jnp.float32),
            pltpu.VMEM((32,), jnp.float32),
            pltpu.SemaphoreType.DMA,
        ],
    )(_lookup_body)

    packed = reformat(op_table.T)
    wb = jnp.concatenate(
        [jnp.zeros((1,), jnp.float32), lin_W.reshape(-1), lin_b,
         jnp.zeros((7,), jnp.float32)])
    idxp = jnp.pad(node_inds.astype(jnp.int32).reshape(nw, bpw // 128, 128),
                   ((0, 0), (0, 8 - bpw // 128), (0, 0)))
    svp = jnp.pad(shape_vals.T, ((0, 8 - N_SHAPE_VALS), (0, 0)))
    tailp = jnp.pad(op_table[VTAIL:], ((0, 0), (0, ROW - OP_D)))
    out_t = lookup(idxp, svp, packed, tailp, wb)
    return out_t.T
